# baseline (speedup 1.0000x reference)
import jax
import jax.numpy as jnp
from jax import lax
from jax.experimental import pallas as pl
from jax.experimental.pallas import tpu as pltpu
from jax.experimental.pallas import tpu_sc as plsc

_NUM_FIELDS = 26
_VOCAB = 100000
_EMBED_DIM = 16
_BATCH = 16384

_NC = 2
_NS = 16
_NW = _NC * _NS

_LANES = 16

_VBLK = 128
_NBLK = (_VOCAB + _VBLK - 1) // _VBLK
_LAST_W = _VOCAB - (_NBLK - 1) * _VBLK
_KMAX = (_NBLK + _NW - 1) // _NW

_ROWS_PER_WORKER = _BATCH // _NW
_CHUNK_ROWS = 128
_CHUNK = _CHUNK_ROWS * _NUM_FIELDS
_NCHUNKS = _ROWS_PER_WORKER // _CHUNK_ROWS


def _detile_body(tt_hbm, flat_hbm, tile_v, obuf, lanes_v):
    wid = lax.axis_index("s") * _NC + lax.axis_index("c")
    lanes_v[...] = lax.iota(jnp.int32, _LANES)

    def field(i, _):
        def task(k, _):
            blk = k * _NW + wid

            @pl.when(blk < _NBLK)
            def _():
                v0 = blk * _VBLK
                pltpu.sync_copy(tt_hbm.at[i, :, pl.ds(v0, _VBLK)], tile_v)
                lanes = lanes_v[...]

                def row(r, _):
                    col = jnp.broadcast_to(r.astype(jnp.int32), (_LANES,))
                    obuf[pl.ds(r * _EMBED_DIM, _EMBED_DIM)] = plsc.load_gather(
                        tile_v, [lanes, col]
                    )
                    return 0

                lax.fori_loop(0, _VBLK, row, 0)
                base = (i * _VOCAB + v0) * _EMBED_DIM

                @pl.when(blk < _NBLK - 1)
                def _():
                    pltpu.sync_copy(
                        obuf.at[pl.ds(0, _VBLK * _EMBED_DIM)],
                        flat_hbm.at[pl.ds(base, _VBLK * _EMBED_DIM)],
                    )

                @pl.when(blk == _NBLK - 1)
                def _():
                    pltpu.sync_copy(
                        obuf.at[pl.ds(0, _LAST_W * _EMBED_DIM)],
                        flat_hbm.at[pl.ds(base, _LAST_W * _EMBED_DIM)],
                    )

            return 0

        lax.fori_loop(0, _KMAX, task, 0)
        return 0

    lax.fori_loop(0, _NUM_FIELDS, field, 0)


def _gather_body(x_hbm, off_hbm, tbl_hbm, out_hbm, raw_v, idx_v, off_v, rows_v,
                 sem):
    wid = lax.axis_index("s") * _NC + lax.axis_index("c")
    pltpu.sync_copy(off_hbm, off_v)

    def chunk(n, _):
        base = (wid * _NCHUNKS + n) * _CHUNK
        pltpu.sync_copy(x_hbm.at[pl.ds(base, _CHUNK)], raw_v)

        def add(j, _):
            s = pl.ds(j * _LANES, _LANES)
            idx_v[s] = raw_v[s] + off_v[s]
            return 0

        lax.fori_loop(0, _CHUNK // _LANES, add, 0)
        pltpu.async_copy(tbl_hbm.at[idx_v], rows_v, sem).wait()
        pltpu.sync_copy(rows_v, out_hbm.at[pl.ds(base, _CHUNK)])
        return 0

    lax.fori_loop(0, _NCHUNKS, chunk, 0)


@jax.jit
def kernel(x, tables):
    mesh = plsc.VectorSubcoreMesh(core_axis_name="c", subcore_axis_name="s")

    tt = jnp.transpose(tables, (0, 2, 1))
    detile = pl.kernel(
        _detile_body,
        out_type=jax.ShapeDtypeStruct(
            (_NUM_FIELDS * _VOCAB * _EMBED_DIM,), jnp.float32
        ),
        mesh=mesh,
        scratch_types=[
            pltpu.VMEM((_EMBED_DIM, _VBLK), jnp.float32),
            pltpu.VMEM((_VBLK * _EMBED_DIM,), jnp.float32),
            pltpu.VMEM((_LANES,), jnp.int32),
        ],
        compiler_params=pltpu.CompilerParams(
            use_tc_tiling_on_sc=True, needs_layout_passes=False
        ),
    )
    flat = detile(tt)

    x_flat = x.astype(jnp.int32).reshape(-1)
    tbl_flat = flat.reshape(_NUM_FIELDS * _VOCAB, _EMBED_DIM)
    offsets = jnp.tile(
        jnp.arange(_NUM_FIELDS, dtype=jnp.int32) * _VOCAB, _CHUNK_ROWS
    )

    gather = pl.kernel(
        _gather_body,
        out_type=jax.ShapeDtypeStruct(
            (_BATCH * _NUM_FIELDS, _EMBED_DIM), jnp.float32
        ),
        mesh=mesh,
        scratch_types=[
            pltpu.VMEM((_CHUNK,), jnp.int32),
            pltpu.VMEM((_CHUNK,), jnp.int32),
            pltpu.VMEM((_CHUNK,), jnp.int32),
            pltpu.VMEM((_CHUNK, _EMBED_DIM), jnp.float32),
            pltpu.SemaphoreType.DMA,
        ],
        compiler_params=pltpu.CompilerParams(use_tc_tiling_on_sc=False),
    )
    out = gather(x_flat, offsets, tbl_flat)
    return out.reshape(_BATCH, _NUM_FIELDS * _EMBED_DIM)
